# Initial kernel scaffold; baseline (speedup 1.0000x reference)
import jax
import jax.numpy as jnp
from jax.experimental import pallas as pl


def kernel(x, edge_index, Wl1, bl1, Wr1, g1, be1, Wl2, bl2, Wr2, g2, be2, Wl3, bl3, Wr3, g3, be3, Wo, bo):
    raise NotImplementedError("write your pallas kernel here")


# speedup vs baseline: 7.6514x; 7.6514x over previous
import functools

import jax
import jax.numpy as jnp
from jax import lax
from jax.experimental import pallas as pl
from jax.experimental.pallas import tpu as pltpu
from jax.experimental.pallas import tpu_sc as plsc

N = 10000
E = 320000
D = 128
NC = 2
NS = 16
NW = NC * NS
CHUNK = 125
RPT = E // (NW * CHUNK)
ZROWS = 40
NZCHUNK = N // ZROWS
DEG_PAD = 10240
DEG_SUB = 1024


def _sc_agg_body(with_deg, h_hbm, srcm, dstm, z2d, z1d, aggp, degp,
                 src_v, dst_v, rows_v, ones_v, zbuf, dzbuf, sem, agg_sh, deg_sh):
    c = lax.axis_index("c")
    s = lax.axis_index("s")
    wid = s * NC + c

    pltpu.sync_copy(z2d, zbuf)
    for k in range((NZCHUNK + NS - 1) // NS):
        m = s + NS * k
        @pl.when(m < NZCHUNK)
        def _():
            pltpu.sync_copy(zbuf, agg_sh.at[pl.ds(m * ZROWS, ZROWS)])
    if with_deg:
        @pl.when(s < DEG_PAD // DEG_SUB)
        def _():
            pltpu.sync_copy(z1d, dzbuf)
            pltpu.sync_copy(dzbuf, deg_sh.at[pl.ds(s * DEG_SUB, DEG_SUB)])
        ones16 = jnp.ones((16,), jnp.float32)
        for k in range(8):
            ones_v[pl.ds(k * 16, 16)] = ones16
    plsc.subcore_barrier()

    pltpu.sync_copy(srcm.at[pl.ds(wid * RPT, RPT)], src_v)
    pltpu.sync_copy(dstm.at[pl.ds(wid * RPT, RPT)], dst_v)

    def step(j, _):
        pltpu.async_copy(h_hbm.at[src_v.at[j]], rows_v, sem).wait()
        pltpu.sync_copy(rows_v, agg_sh.at[dst_v.at[j]], add=True)
        if with_deg:
            pltpu.sync_copy(ones_v.at[pl.ds(0, CHUNK)], deg_sh.at[dst_v.at[j]],
                            add=True)
        return 0

    lax.fori_loop(0, RPT, step, 0)
    plsc.subcore_barrier()

    for k in range((NZCHUNK + NS - 1) // NS):
        m = s + NS * k
        @pl.when(m < NZCHUNK)
        def _():
            pltpu.sync_copy(agg_sh.at[pl.ds(m * ZROWS, ZROWS)], zbuf)
            pltpu.sync_copy(zbuf, aggp.at[c].at[pl.ds(m * ZROWS, ZROWS)])
    if with_deg:
        @pl.when(s < DEG_PAD // DEG_SUB)
        def _():
            pltpu.sync_copy(deg_sh.at[pl.ds(s * DEG_SUB, DEG_SUB)], dzbuf)
            pltpu.sync_copy(dzbuf,
                            degp.at[pl.ds(c * DEG_PAD + s * DEG_SUB, DEG_SUB)])


def _make_sc_agg(with_deg):
    mesh = plsc.VectorSubcoreMesh(core_axis_name="c", subcore_axis_name="s")
    out_type = (jax.ShapeDtypeStruct((NC, N, D), jnp.float32),
                jax.ShapeDtypeStruct((NC * DEG_PAD,), jnp.float32))
    scratch = [
        pltpu.VMEM((RPT, CHUNK), jnp.int32),
        pltpu.VMEM((RPT, CHUNK), jnp.int32),
        pltpu.VMEM((CHUNK, D), jnp.float32),
        pltpu.VMEM((128,), jnp.float32),
        pltpu.VMEM((ZROWS, D), jnp.float32),
        pltpu.VMEM((DEG_SUB,), jnp.float32),
        pltpu.SemaphoreType.DMA,
        pltpu.VMEM_SHARED((N, D), jnp.float32),
        pltpu.VMEM_SHARED((DEG_PAD,), jnp.float32),
    ]
    body = functools.partial(_sc_agg_body, with_deg)
    return pl.kernel(body, out_type=out_type, mesh=mesh, scratch_types=scratch,
                     name="sc_agg_deg" if with_deg else "sc_agg")


_sc_agg_with_deg = _make_sc_agg(True)
_sc_agg_plain = _make_sc_agg(False)

TCR = 2000


def _tc_layer1_body(aggA, aggB, degA, degB, x, Wl, Wr, b, h_out, inv_out):
    deg = jnp.maximum(degA[...] + degB[...], 1.0)
    inv = 1.0 / deg
    mean = (aggA[...] + aggB[...]) * inv
    h = jnp.dot(mean, Wl[...], preferred_element_type=jnp.float32)
    h += jnp.dot(x[...], Wr[...], preferred_element_type=jnp.float32)
    h += b[...]
    h_out[...] = jnp.maximum(h, 0.0)
    inv_out[...] = inv


def _tc_layer_body(has_head, aggA, aggB, inv, hp, Wl, Wr, b, *rest):
    mean = (aggA[...] + aggB[...]) * inv[...]
    h = jnp.dot(mean, Wl[...], preferred_element_type=jnp.float32)
    h += jnp.dot(hp[...], Wr[...], preferred_element_type=jnp.float32)
    h += b[...]
    h = jnp.maximum(h, 0.0) + hp[...]
    if has_head:
        Wo, bo, out = rest
        out[...] = jnp.dot(h, Wo[...], preferred_element_type=jnp.float32) + bo[...]
    else:
        (out,) = rest
        out[...] = h


_row_spec = pl.BlockSpec((TCR, D), lambda i: (i, 0))
_col_spec = pl.BlockSpec((TCR, 1), lambda i: (i, 0))
_w_spec = pl.BlockSpec((D, D), lambda i: (0, 0))
_b_spec = pl.BlockSpec((1, D), lambda i: (0, 0))

_tc_layer1 = pl.pallas_call(
    _tc_layer1_body,
    grid=(N // TCR,),
    in_specs=[_row_spec, _row_spec, _col_spec, _col_spec, _row_spec,
              _w_spec, _w_spec, _b_spec],
    out_specs=[_row_spec, _col_spec],
    out_shape=[jax.ShapeDtypeStruct((N, D), jnp.float32),
               jax.ShapeDtypeStruct((N, 1), jnp.float32)],
)

_tc_layer_mid = pl.pallas_call(
    functools.partial(_tc_layer_body, False),
    grid=(N // TCR,),
    in_specs=[_row_spec, _row_spec, _col_spec, _row_spec,
              _w_spec, _w_spec, _b_spec],
    out_specs=_row_spec,
    out_shape=jax.ShapeDtypeStruct((N, D), jnp.float32),
)

_tc_layer_last = pl.pallas_call(
    functools.partial(_tc_layer_body, True),
    grid=(N // TCR,),
    in_specs=[_row_spec, _row_spec, _col_spec, _row_spec,
              _w_spec, _w_spec, _b_spec, _w_spec, _b_spec],
    out_specs=_row_spec,
    out_shape=jax.ShapeDtypeStruct((N, D), jnp.float32),
)


def kernel(x, edge_index, Wl1, bl1, Wr1, g1, be1, Wl2, bl2, Wr2, g2, be2,
           Wl3, bl3, Wr3, g3, be3, Wo, bo):
    src = edge_index[0].reshape(NW * RPT, CHUNK)
    dst = edge_index[1].reshape(NW * RPT, CHUNK)
    z2d = jnp.zeros((ZROWS, D), jnp.float32)
    z1d = jnp.zeros((DEG_SUB,), jnp.float32)

    def fold(Wl, bl, Wr, g, be):
        s = (g / jnp.sqrt(1.0 + 1e-5))[None, :]
        return Wl * s, Wr * s, (bl[None, :] * s + be[None, :])

    Wl1f, Wr1f, b1f = fold(Wl1, bl1, Wr1, g1, be1)
    Wl2f, Wr2f, b2f = fold(Wl2, bl2, Wr2, g2, be2)
    Wl3f, Wr3f, b3f = fold(Wl3, bl3, Wr3, g3, be3)

    aggp, degp = _sc_agg_with_deg(x, src, dst, z2d, z1d)
    degA = degp[0:N, None]
    degB = degp[DEG_PAD:DEG_PAD + N, None]
    h1, inv = _tc_layer1(aggp[0], aggp[1], degA, degB, x, Wl1f, Wr1f, b1f)
    aggp2, _ = _sc_agg_plain(h1, src, dst, z2d, z1d)
    h2 = _tc_layer_mid(aggp2[0], aggp2[1], inv, h1, Wl2f, Wr2f, b2f)
    aggp3, _ = _sc_agg_plain(h2, src, dst, z2d, z1d)
    return _tc_layer_last(aggp3[0], aggp3[1], inv, h2, Wl3f, Wr3f, b3f,
                          Wo, bo[None, :])
